# baseline (speedup 1.0000x reference)
import functools

import jax
import jax.numpy as jnp
import numpy as np
from jax import lax
from jax.experimental import pallas as pl
from jax.experimental.pallas import tpu as pltpu
from jax.experimental.pallas import tpu_sc as plsc

EMB_DIM = 128
SEQ = 200
SCALE = float(np.sqrt(float(EMB_DIM)))

NUM_CORES = 2
NUM_SUBCORES = 16
NUM_WORKERS = NUM_CORES * NUM_SUBCORES
CHUNK = 128
VEC = 16
GRPS = EMB_DIM // VEC


def _pos_table(max_length, emb_dim):
    pos = np.arange(max_length)[:, np.newaxis]
    div_term = np.exp(np.arange(0, emb_dim, 2) * -(np.log(10000.0) / emb_dim))
    pos_emb = pos * div_term
    pos_emb = np.stack([np.sin(pos_emb), np.cos(pos_emb)], axis=1).reshape(max_length, -1)
    pos_emb[1:, 1::2] = 0
    return pos_emb.astype(np.float32)


_POS = _pos_table(SEQ + 1, EMB_DIM)[:SEQ]
PSEQ = 256
_POSP = np.concatenate([_POS, np.zeros((PSEQ - SEQ, EMB_DIM), np.float32)], axis=0)


def _encode_body(idx_hbm, sidx_hbm, emb_hbm, seg_hbm, pos_hbm, out_hbm,
                 idx_all, sidx_all, aidx_all, comb_sh, segv, rows, addbuf, obuf,
                 gsem, gsemb, gsemc, gsemd, asem, osem, rows_per_w, n_chunks):
    wid = lax.axis_index("s") * NUM_CORES + lax.axis_index("c")
    wbase = wid * rows_per_w

    pltpu.sync_copy(idx_hbm.at[pl.ds(wbase, rows_per_w)], idx_all)
    pltpu.sync_copy(sidx_hbm.at[pl.ds(wbase, rows_per_w)], sidx_all)
    pltpu.sync_copy(seg_hbm, segv)

    tid = lax.axis_index("s")
    BUILD = 2 * PSEQ // NUM_SUBCORES
    half = tid // (NUM_SUBCORES // 2)
    l0 = pl.multiple_of(lax.rem(tid, NUM_SUBCORES // 2) * BUILD, 8)
    pltpu.sync_copy(pos_hbm.at[pl.ds(l0, BUILD)], rows[0].at[pl.ds(0, BUILD)])

    @plsc.parallel_loop(0, BUILD)
    def _(r):
        for g in range(GRPS):
            sl = pl.ds(g * VEC, VEC)
            rows[0][r, sl] = rows[0][r, sl] + segv[half, sl]

    pltpu.sync_copy(rows[0].at[pl.ds(0, BUILD)],
                    comb_sh.at[pl.ds(pl.multiple_of(tid * BUILD, 8), BUILD)])
    plsc.subcore_barrier()

    @plsc.parallel_loop(0, rows_per_w // VEC, unroll=2)
    def _(v):
        base = v * VEC
        l16 = lax.rem(base + lax.iota(jnp.int32, VEC), SEQ)
        aidx_all[pl.ds(base, VEC)] = sidx_all[pl.ds(base, VEC)] * PSEQ + l16

    NSTR = 2
    HCH = CHUNK // NSTR
    gsems = [gsem, gsemb]

    asems = [asem, gsemc]

    def issue(s, ci):
        off = ci * CHUNK
        for k in range(NSTR):
            pltpu.async_copy(emb_hbm.at[idx_all.at[pl.ds(off + k * HCH, HCH)]],
                             rows[s].at[pl.ds(k * HCH, HCH)], gsems[k][s])
            pltpu.async_copy(comb_sh.at[aidx_all.at[pl.ds(off + k * HCH, HCH)]],
                             addbuf[s].at[pl.ds(k * HCH, HCH)], asems[k][s])

    def wait_gathers(s, ci):
        off = ci * CHUNK
        for k in range(NSTR):
            pltpu.make_async_copy(emb_hbm.at[idx_all.at[pl.ds(off + k * HCH, HCH)]],
                                  rows[s].at[pl.ds(k * HCH, HCH)], gsems[k][s]).wait()
            pltpu.make_async_copy(comb_sh.at[aidx_all.at[pl.ds(off + k * HCH, HCH)]],
                                  addbuf[s].at[pl.ds(k * HCH, HCH)], asems[k][s]).wait()

    def out_start(s, ci):
        base = wbase + ci * CHUNK
        pltpu.async_copy(obuf[s], out_hbm.at[pl.ds(base, CHUNK)], osem[s])

    def out_wait(s, ci):
        base = wbase + ci * CHUNK
        pltpu.make_async_copy(obuf[s], out_hbm.at[pl.ds(base, CHUNK)], osem[s]).wait()

    def compute(s, ci):
        @plsc.parallel_loop(0, CHUNK, unroll=4)
        def _(r):
            for g in range(GRPS):
                sl = pl.ds(g * VEC, VEC)
                obuf[s][r, sl] = rows[s][r, sl] * SCALE + addbuf[s][r, sl]

    issue(0, 0)
    issue(1, 1)

    def chunk_pair(ci2, carry):
        for s in (0, 1):
            ci = ci2 * 2 + s
            wait_gathers(s, ci)

            @pl.when(ci2 >= 1)
            def _():
                out_wait(s, ci - 2)

            compute(s, ci)

            @pl.when(ci + 2 < n_chunks)
            def _():
                issue(s, ci + 2)

            out_start(s, ci)
        return carry

    lax.fori_loop(0, n_chunks // 2, chunk_pair, 0, unroll=False)
    out_wait(0, n_chunks - 2)
    out_wait(1, n_chunks - 1)


def kernel(batch, segment_ids, emb_table, seg_table):
    B, L = batch.shape
    N = B * L
    rows_per_w = N // NUM_WORKERS
    n_chunks = rows_per_w // CHUNK

    idx = batch.reshape(N).astype(jnp.int32)
    sidx = segment_ids.reshape(N).astype(jnp.int32)
    pos = jnp.asarray(_POSP)

    body = functools.partial(_encode_body, rows_per_w=rows_per_w, n_chunks=n_chunks)
    run = pl.kernel(
        body,
        out_type=jax.ShapeDtypeStruct((N, EMB_DIM), jnp.float32),
        mesh=plsc.VectorSubcoreMesh(
            core_axis_name="c", subcore_axis_name="s",
            num_cores=NUM_CORES, num_subcores=NUM_SUBCORES),
        scratch_types=[
            pltpu.VMEM((rows_per_w,), jnp.int32),
            pltpu.VMEM((rows_per_w,), jnp.int32),
            pltpu.VMEM((rows_per_w,), jnp.int32),
            pltpu.VMEM_SHARED((2 * PSEQ, EMB_DIM), jnp.float32),
            pltpu.VMEM((2, EMB_DIM), jnp.float32),
            [pltpu.VMEM((CHUNK, EMB_DIM), jnp.float32) for _ in range(2)],
            [pltpu.VMEM((CHUNK, EMB_DIM), jnp.float32) for _ in range(2)],
            [pltpu.VMEM((CHUNK, EMB_DIM), jnp.float32) for _ in range(2)],
            [pltpu.SemaphoreType.DMA for _ in range(2)],
            [pltpu.SemaphoreType.DMA for _ in range(2)],
            [pltpu.SemaphoreType.DMA for _ in range(2)],
            [pltpu.SemaphoreType.DMA for _ in range(2)],
            [pltpu.SemaphoreType.DMA for _ in range(2)],
            [pltpu.SemaphoreType.DMA for _ in range(2)],
        ],
    )
    out = run(idx, sidx, emb_table, seg_table, pos)
    return out.reshape(B, L, EMB_DIM)
